# Initial kernel scaffold; baseline (speedup 1.0000x reference)
import jax
import jax.numpy as jnp
from jax.experimental import pallas as pl


def kernel(x, edge_index, batch, Wl0, bl0, Wr0, Wl1, bl1, Wr1, Wl2, bl2, Wr2, W_fc, b_fc):
    raise NotImplementedError("write your pallas kernel here")


# speedup vs baseline: 3.3602x; 3.3602x over previous
import functools

import jax
import jax.numpy as jnp
from jax import lax
from jax.experimental import pallas as pl
from jax.experimental.pallas import tpu as pltpu
from jax.experimental.pallas import tpu_sc as plsc

F32 = jnp.float32
I32 = jnp.int32
HI = lax.Precision.HIGHEST

N = 10000
G = 16
TILES = 16
CORES = 2
CH = 128
DH = 128
N_ACC = 10240
ROWS_PT = N_ACC // TILES
CPT_FEAT = 158
E_PAD = CPT_FEAT * CH * TILES
CPT_EDGE = E_PAD // (CH * TILES * CORES)
BR = 2000


def _sc_agg_feat(t_a, t_b, src_p, dst_p, zeros_d):
    mesh = plsc.VectorSubcoreMesh(core_axis_name="c", subcore_axis_name="s")

    @functools.partial(
        pl.kernel,
        out_type=jax.ShapeDtypeStruct((CORES * N_ACC, DH), F32),
        mesh=mesh,
        scratch_types=[
            pltpu.VMEM((CH,), I32),
            pltpu.VMEM((CH,), I32),
            pltpu.VMEM((CH, DH), F32),
            pltpu.VMEM_SHARED((N_ACC, DH), F32),
            pltpu.SemaphoreType.DMA,
        ],
    )
    def k(ta_hbm, tb_hbm, src_hbm, dst_hbm, z_hbm, out_hbm,
          src_v, dst_v, rows_v, acc, sem):
        c = lax.axis_index("c")
        s = lax.axis_index("s")
        r0 = s * ROWS_PT
        pltpu.sync_copy(z_hbm.at[pl.ds(r0, ROWS_PT)], acc.at[pl.ds(r0, ROWS_PT)])
        plsc.subcore_barrier()
        base = s * (CPT_FEAT * CH)

        def body(j, carry):
            b = base + j * CH
            pltpu.sync_copy(src_hbm.at[pl.ds(b, CH)], src_v)
            pltpu.sync_copy(dst_hbm.at[pl.ds(b, CH)], dst_v)

            @pl.when(c == 0)
            def _():
                pltpu.async_copy(ta_hbm.at[src_v], rows_v, sem).wait()

            @pl.when(c == 1)
            def _():
                pltpu.async_copy(tb_hbm.at[src_v], rows_v, sem).wait()

            pltpu.sync_copy(rows_v, acc.at[dst_v], add=True)
            return carry

        lax.fori_loop(0, CPT_FEAT, body, 0)
        plsc.subcore_barrier()
        pltpu.sync_copy(acc.at[pl.ds(r0, ROWS_PT)],
                        out_hbm.at[pl.ds(c * N_ACC + r0, ROWS_PT)])

    return k(t_a, t_b, src_p, dst_p, zeros_d)


def _sc_agg_edge(t, src_p, dst_p, zeros_d):
    mesh = plsc.VectorSubcoreMesh(core_axis_name="c", subcore_axis_name="s")

    @functools.partial(
        pl.kernel,
        out_type=jax.ShapeDtypeStruct((CORES * N_ACC, DH), F32),
        mesh=mesh,
        scratch_types=[
            pltpu.VMEM((CH,), I32),
            pltpu.VMEM((CH,), I32),
            pltpu.VMEM((CH, DH), F32),
            pltpu.VMEM_SHARED((N_ACC, DH), F32),
            pltpu.SemaphoreType.DMA,
        ],
    )
    def k(t_hbm, src_hbm, dst_hbm, z_hbm, out_hbm,
          src_v, dst_v, rows_v, acc, sem):
        c = lax.axis_index("c")
        s = lax.axis_index("s")
        r0 = s * ROWS_PT
        pltpu.sync_copy(z_hbm.at[pl.ds(r0, ROWS_PT)], acc.at[pl.ds(r0, ROWS_PT)])
        plsc.subcore_barrier()
        base = (c * TILES + s) * (CPT_EDGE * CH)

        def body(j, carry):
            b = base + j * CH
            pltpu.sync_copy(src_hbm.at[pl.ds(b, CH)], src_v)
            pltpu.sync_copy(dst_hbm.at[pl.ds(b, CH)], dst_v)
            pltpu.async_copy(t_hbm.at[src_v], rows_v, sem).wait()
            pltpu.sync_copy(rows_v, acc.at[dst_v], add=True)
            return carry

        lax.fori_loop(0, CPT_EDGE, body, 0)
        plsc.subcore_barrier()
        pltpu.sync_copy(acc.at[pl.ds(r0, ROWS_PT)],
                        out_hbm.at[pl.ds(c * N_ACC + r0, ROWS_PT)])

    return k(t, src_p, dst_p, zeros_d)


def _sc_count(dst_p, zeros_d, ones_ch):
    mesh = plsc.VectorSubcoreMesh(core_axis_name="c", subcore_axis_name="s")

    @functools.partial(
        pl.kernel,
        out_type=jax.ShapeDtypeStruct((CORES * N_ACC, DH), F32),
        mesh=mesh,
        scratch_types=[
            pltpu.VMEM((CH,), I32),
            pltpu.VMEM((CH, DH), F32),
            pltpu.VMEM_SHARED((N_ACC, DH), F32),
        ],
    )
    def k(dst_hbm, z_hbm, ones_hbm, out_hbm, dst_v, ones_v, acc):
        c = lax.axis_index("c")
        s = lax.axis_index("s")
        r0 = s * ROWS_PT
        pltpu.sync_copy(ones_hbm, ones_v)
        pltpu.sync_copy(z_hbm.at[pl.ds(r0, ROWS_PT)], acc.at[pl.ds(r0, ROWS_PT)])
        plsc.subcore_barrier()
        base = (c * TILES + s) * (CPT_EDGE * CH)

        def body(j, carry):
            pltpu.sync_copy(dst_hbm.at[pl.ds(base + j * CH, CH)], dst_v)
            pltpu.sync_copy(ones_v, acc.at[dst_v], add=True)
            return carry

        lax.fori_loop(0, CPT_EDGE, body, 0)
        plsc.subcore_barrier()
        pltpu.sync_copy(acc.at[pl.ds(r0, ROWS_PT)],
                        out_hbm.at[pl.ds(c * N_ACC + r0, ROWS_PT)])

    return k(dst_p, zeros_d, ones_ch)


def _tc_layer(agg_a, agg_b, cnt_a, cnt_b, hs, wl_parts, wr_parts, bl2):
    nh = len(hs)

    def body(*refs):
        agg_a_r, agg_b_r, cnt_a_r, cnt_b_r = refs[:4]
        h_rs = refs[4:4 + nh]
        wl_a_r, wl_b_r = refs[4 + nh], refs[5 + nh]
        wr_rs = refs[6 + nh:6 + 2 * nh]
        bl_r = refs[6 + 2 * nh]
        out_r = refs[-1]
        cnt = cnt_a_r[:, :1] + cnt_b_r[:, :1]
        inv = 1.0 / jnp.maximum(cnt, 1.0)
        acc = jnp.dot(agg_a_r[...] * inv, wl_a_r[...],
                      preferred_element_type=F32, precision=HI)
        acc = acc + jnp.dot(agg_b_r[...] * inv, wl_b_r[...],
                            preferred_element_type=F32, precision=HI)
        for h_r, wr_r in zip(h_rs, wr_rs):
            acc = acc + jnp.dot(h_r[...], wr_r[...],
                                preferred_element_type=F32, precision=HI)
        h = jnp.maximum(acc + bl_r[...], 0.0)
        out_r[0] = h[:, :128]
        out_r[1] = h[:, 128:]

    in_specs = (
        [pl.BlockSpec((BR, DH), lambda i: (i, 0)),
         pl.BlockSpec((BR, DH), lambda i: (i, 0)),
         pl.BlockSpec((BR, DH), lambda i: (i, 0)),
         pl.BlockSpec((BR, DH), lambda i: (i, 0))]
        + [pl.BlockSpec((BR, h.shape[1]), lambda i: (i, 0)) for h in hs]
        + [pl.BlockSpec(w.shape, lambda i: (0, 0)) for w in wl_parts]
        + [pl.BlockSpec(w.shape, lambda i: (0, 0)) for w in wr_parts]
        + [pl.BlockSpec(bl2.shape, lambda i: (0, 0))]
    )
    return pl.pallas_call(
        body,
        grid=(N // BR,),
        in_specs=in_specs,
        out_specs=pl.BlockSpec((2, BR, 128), lambda i: (0, i, 0)),
        out_shape=jax.ShapeDtypeStruct((2, N, 128), F32),
    )(agg_a, agg_b, cnt_a, cnt_b, *hs, *wl_parts, *wr_parts, bl2)


def _tc_final(h_a, h_b, batch3, w_a, w_b, b_tile):
    def body(ha_r, hb_r, b_r, wa_r, wb_r, bt_r, out_r):
        i = pl.program_id(0)

        @pl.when(i == 0)
        def _():
            out_r[...] = bt_r[...]

        y = (jnp.dot(ha_r[...], wa_r[...], preferred_element_type=F32, precision=HI)
             + jnp.dot(hb_r[...], wb_r[...], preferred_element_type=F32, precision=HI))
        m = (b_r[0] == lax.broadcasted_iota(I32, (G, BR), 0)).astype(F32)
        out_r[...] += jnp.dot(m, y, preferred_element_type=F32, precision=HI)

    return pl.pallas_call(
        body,
        grid=(N // BR,),
        in_specs=[
            pl.BlockSpec((BR, 128), lambda i: (i, 0)),
            pl.BlockSpec((BR, 128), lambda i: (i, 0)),
            pl.BlockSpec((1, 1, BR), lambda i: (i, 0, 0)),
            pl.BlockSpec((128, 128), lambda i: (0, 0)),
            pl.BlockSpec((128, 128), lambda i: (0, 0)),
            pl.BlockSpec((G, 128), lambda i: (0, 0)),
        ],
        out_specs=pl.BlockSpec((G, 128), lambda i: (0, 0)),
        out_shape=jax.ShapeDtypeStruct((G, 128), F32),
    )(h_a, h_b, batch3, w_a, w_b, b_tile)


def kernel(x, edge_index, batch, Wl0, bl0, Wr0, Wl1, bl1, Wr1, Wl2, bl2, Wr2,
           W_fc, b_fc):
    src = edge_index[0].astype(I32)
    dst = edge_index[1].astype(I32)
    e = src.shape[0]
    pad = E_PAD - e
    src_p = jnp.concatenate([src, jnp.zeros((pad,), I32)])
    dst_p = jnp.concatenate([dst, jnp.full((pad,), N, I32)])

    zeros_d = jnp.zeros((N_ACC, DH), F32)
    ones_ch = jnp.ones((CH, DH), F32)

    cnt = _sc_count(dst_p, zeros_d, ones_ch)
    cnt_a, cnt_b = cnt[:N_ACC], cnt[N_ACC:]

    agg0 = _sc_agg_edge(x, src_p, dst_p, zeros_d)
    h1 = _tc_layer(agg0[:N_ACC], agg0[N_ACC:], cnt_a, cnt_b, [x],
                   [Wl0, Wl0], [Wr0], bl0.reshape(1, -1))

    agg1 = _sc_agg_feat(h1[0], h1[1], src_p, dst_p, zeros_d)
    h2 = _tc_layer(agg1[:N_ACC], agg1[N_ACC:], cnt_a, cnt_b, [h1[0], h1[1]],
                   [Wl1[:128], Wl1[128:]], [Wr1[:128], Wr1[128:]],
                   bl1.reshape(1, -1))

    agg2 = _sc_agg_feat(h2[0], h2[1], src_p, dst_p, zeros_d)
    h3 = _tc_layer(agg2[:N_ACC], agg2[N_ACC:], cnt_a, cnt_b, [h2[0], h2[1]],
                   [Wl2[:128], Wl2[128:]], [Wr2[:128], Wr2[128:]],
                   bl2.reshape(1, -1))

    w_a = jnp.zeros((128, 128), F32).at[:, :1].set(W_fc[:128])
    w_b = jnp.zeros((128, 128), F32).at[:, :1].set(W_fc[128:])
    b_tile = jnp.zeros((G, 128), F32).at[:, 0].set(b_fc[0])
    batch3 = batch.astype(I32).reshape(N // BR, 1, BR)

    out = _tc_final(h3[0], h3[1], batch3, w_a, w_b, b_tile)
    return out[:, :1]
